# baseline (device time: 898704 ns/iter reference)
import jax
import jax.numpy as jnp
from jax import lax
from jax.experimental import pallas as pl
from jax.experimental.pallas import tpu as pltpu

T, D, V = 2048, 4096, 16384
RB = T // 4
CB = V // 2

S_STATS, S_Z, S_YA, S_YB, S_XA, S_XB, S_XCA, S_XCB = range(8)


def kernel(x, W):
    xi = lax.axis_index("x")
    yi = lax.axis_index("y")
    zi = lax.axis_index("z")
    r = 2 * xi + zi
    x_rows = lax.dynamic_slice_in_dim(x, r * RB, RB, axis=0)
    logits = jnp.dot(x_rows.astype(jnp.bfloat16), W.astype(jnp.bfloat16),
                     preferred_element_type=jnp.float32)

    def body(logits_ref, out_ref, l_ref, e_ref, st_send, st_recv,
             send_sems, recv_sems, copy_sem, in_sem):
        xi = lax.axis_index("x")
        yi = lax.axis_index("y")
        zi = lax.axis_index("z")
        row0 = (2 * xi + zi) * RB
        rowz = (2 * xi + (1 - zi)) * RB
        col0 = yi * CB
        colo = (1 - yi) * CB
        z_peer = (xi, yi, 1 - zi)
        y_peer = (xi, 1 - yi, zi)
        x_peer = (1 - xi, yi, zi)

        def rdma(src, dst, sem_idx, peer):
            return pltpu.make_async_remote_copy(
                src_ref=src, dst_ref=dst,
                send_sem=send_sems.at[sem_idx], recv_sem=recv_sems.at[sem_idx],
                device_id=peer, device_id_type=pl.DeviceIdType.MESH)

        load = pltpu.make_async_copy(logits_ref, l_ref, in_sem)
        load.start()

        barrier = pltpu.get_barrier_semaphore()
        for nbr in [x_peer, y_peer, z_peer]:
            pl.semaphore_signal(barrier, inc=1, device_id=nbr,
                                device_id_type=pl.DeviceIdType.MESH)
        pl.semaphore_wait(barrier, 3)
        load.wait()

        l = l_ref[:, :]
        m = jnp.max(l, axis=1, keepdims=True)
        e = jnp.exp(l - m)
        e_ref[:, :] = e
        s = jnp.sum(e, axis=1, keepdims=True)
        st_send[:, 0:128] = jnp.broadcast_to(m, (RB, 128))
        st_send[:, 128:256] = jnp.broadcast_to(s, (RB, 128))

        stats_rdma = rdma(st_send, st_recv, S_STATS, y_peer)
        stats_rdma.start()
        stats_rdma.wait()

        m_o = st_recv[:, 0:1]
        s_o = st_recv[:, 128:129]
        m_g = jnp.maximum(m, m_o)
        s_g = s * jnp.exp(m - m_g) + s_o * jnp.exp(m_o - m_g)
        e_ref[:, :] = e_ref[:, :] * (jnp.exp(m - m_g) / s_g)

        local = pltpu.make_async_copy(
            e_ref, out_ref.at[pl.ds(row0, RB), pl.ds(col0, CB)], copy_sem)
        local.start()
        z_a = rdma(e_ref, out_ref.at[pl.ds(row0, RB), pl.ds(col0, CB)],
                   S_Z, z_peer)
        y_a = rdma(e_ref, out_ref.at[pl.ds(row0, RB), pl.ds(col0, CB)],
                   S_YA, y_peer)
        x_a = rdma(e_ref, out_ref.at[pl.ds(row0, RB), pl.ds(col0, CB)],
                   S_XA, x_peer)
        z_a.start()
        y_a.start()
        x_a.start()

        z_a.wait_recv()
        y_b = rdma(out_ref.at[pl.ds(rowz, RB), pl.ds(col0, CB)],
                   out_ref.at[pl.ds(rowz, RB), pl.ds(col0, CB)],
                   S_YB, y_peer)
        x_b = rdma(out_ref.at[pl.ds(rowz, RB), pl.ds(col0, CB)],
                   out_ref.at[pl.ds(rowz, RB), pl.ds(col0, CB)],
                   S_XB, x_peer)
        y_b.start()
        x_b.start()

        y_a.wait_recv()
        x_ca = rdma(out_ref.at[pl.ds(row0, RB), pl.ds(colo, CB)],
                    out_ref.at[pl.ds(row0, RB), pl.ds(colo, CB)],
                    S_XCA, x_peer)
        x_ca.start()

        y_b.wait_recv()
        x_cb = rdma(out_ref.at[pl.ds(rowz, RB), pl.ds(colo, CB)],
                    out_ref.at[pl.ds(rowz, RB), pl.ds(colo, CB)],
                    S_XCB, x_peer)
        x_cb.start()

        x_a.wait_recv()
        x_b.wait_recv()
        x_ca.wait_recv()
        x_cb.wait_recv()
        local.wait()
        z_a.wait_send()
        y_a.wait_send()
        y_b.wait_send()
        x_a.wait_send()
        x_b.wait_send()
        x_ca.wait_send()
        x_cb.wait_send()

    return pl.pallas_call(
        body,
        out_shape=jax.ShapeDtypeStruct((T, V), jnp.float32),
        in_specs=[pl.BlockSpec(memory_space=pl.ANY)],
        out_specs=pl.BlockSpec(memory_space=pl.ANY),
        scratch_shapes=[
            pltpu.VMEM((RB, CB), jnp.float32),
            pltpu.VMEM((RB, CB), jnp.float32),
            pltpu.VMEM((RB, 256), jnp.float32),
            pltpu.VMEM((RB, 256), jnp.float32),
            pltpu.SemaphoreType.DMA((8,)),
            pltpu.SemaphoreType.DMA((8,)),
            pltpu.SemaphoreType.DMA,
            pltpu.SemaphoreType.DMA,
        ],
        compiler_params=pltpu.CompilerParams(
            collective_id=0, vmem_limit_bytes=60 * 1024 * 1024),
    )(logits)


# device time: 890080 ns/iter; 1.0097x vs baseline; 1.0097x over previous
import jax
import jax.numpy as jnp
from jax import lax
from jax.experimental import pallas as pl
from jax.experimental.pallas import tpu as pltpu

jax.config.update("jax_compilation_cache_dir", "/tmp/jax_comp_cache")
jax.config.update("jax_persistent_cache_min_compile_time_secs", 1.0)

T, D, V = 2048, 4096, 16384
RB = T // 4
CB = V // 2
NC = 16
WC = CB // NC
PB = 8
NQ = 4096


def kernel(x, W):
    xi = lax.axis_index("x")
    zi = lax.axis_index("z")
    x_rows = lax.dynamic_slice_in_dim(x, (2 * xi + zi) * RB, RB, axis=0)

    def body(x_ref, w_ref, out_ref, wbuf, piece, stage, st_mine, st_peer,
             st_all, wsems, lsems, zsend, zrecv, ysend, yrecv, xsend, xrecv,
             fsend, frecv, ssend, srecv, stage_sem):
        xi = lax.axis_index("x")
        yi = lax.axis_index("y")
        zi = lax.axis_index("z")
        row0 = (2 * xi + zi) * RB
        rowz = (2 * xi + (1 - zi)) * RB
        rowx0 = (2 * (1 - xi) + zi) * RB
        xrow0 = xi * (2 * RB)
        col0 = yi * CB
        colo = (1 - yi) * CB
        z_peer = (xi, yi, 1 - zi)
        y_peer = (xi, 1 - yi, zi)
        x_peer = (1 - xi, yi, zi)

        def rdma(src, dst, ssem, rsem, peer):
            return pltpu.make_async_remote_copy(
                src_ref=src, dst_ref=dst, send_sem=ssem, recv_sem=rsem,
                device_id=peer, device_id_type=pl.DeviceIdType.MESH)

        def wload(c):
            return pltpu.make_async_copy(
                w_ref.at[:, pl.ds(c * WC, WC)], wbuf.at[c % 2],
                wsems.at[c % 2])

        wload(0).start()

        barrier = pltpu.get_barrier_semaphore()
        for nbr in [x_peer, y_peer, z_peer]:
            pl.semaphore_signal(barrier, inc=1, device_id=nbr,
                                device_id_type=pl.DeviceIdType.MESH)
        pl.semaphore_wait(barrier, 3)

        x_val = x_ref[:, :]
        z_h = [None] * NC
        y_h = [None] * NC
        x_h = [None] * NC
        l_h = [None] * NC
        m_run = None
        s_run = None

        for c in range(NC):
            pltpu.make_async_copy(
                w_ref.at[:, pl.ds(c * WC, WC)], wbuf.at[c % 2],
                wsems.at[c % 2]).wait()
            if c + 1 < NC:
                wload(c + 1).start()
            if c >= PB:
                z_h[c - PB].wait_send()
                y_h[c - PB].wait_send()
                x_h[c - PB].wait_send()
                l_h[c - PB].wait()
            l_c = jnp.dot(x_val, wbuf[c % 2, :, :],
                          preferred_element_type=jnp.float32)
            pb = c % PB
            piece[pb, :, :] = l_c
            mc = jnp.max(l_c, axis=1, keepdims=True)
            if c == 0:
                m_run = mc
                s_run = jnp.sum(jnp.exp(l_c - mc), axis=1, keepdims=True)
            else:
                m_new = jnp.maximum(m_run, mc)
                s_run = (s_run * jnp.exp(m_run - m_new)
                         + jnp.sum(jnp.exp(l_c - m_new), axis=1, keepdims=True))
                m_run = m_new
            dst = out_ref.at[pl.ds(row0, RB), pl.ds(col0 + c * WC, WC)]
            l_h[c] = pltpu.make_async_copy(piece.at[pb], dst, lsems.at[c])
            l_h[c].start()
            z_h[c] = rdma(piece.at[pb], dst, zsend.at[c], zrecv.at[c], z_peer)
            y_h[c] = rdma(piece.at[pb], dst, ysend.at[c], yrecv.at[c], y_peer)
            x_h[c] = rdma(piece.at[pb], dst, xsend.at[c], xrecv.at[c], x_peer)
            z_h[c].start()
            y_h[c].start()
            x_h[c].start()

        for c in range(NC - PB, NC):
            z_h[c].wait_send()
            y_h[c].wait_send()
            x_h[c].wait_send()
            l_h[c].wait()

        st_mine[:, 0:128] = jnp.broadcast_to(m_run, (RB, 128))
        st_mine[:, 128:256] = jnp.broadcast_to(s_run, (RB, 128))
        sy = rdma(st_mine, st_peer, ssend.at[0], srecv.at[0], y_peer)
        sy.start()
        sy.wait()
        m_o = st_peer[:, 0:1]
        s_o = st_peer[:, 128:129]
        m_g = jnp.maximum(m_run, m_o)
        s_g = s_run * jnp.exp(m_run - m_g) + s_o * jnp.exp(m_o - m_g)
        st_all[pl.ds(row0, RB), 0:128] = jnp.broadcast_to(m_g, (RB, 128))
        st_all[pl.ds(row0, RB), 128:256] = jnp.broadcast_to(1.0 / s_g, (RB, 128))
        sz = rdma(st_all.at[pl.ds(row0, RB)], st_all.at[pl.ds(row0, RB)],
                  ssend.at[1], srecv.at[1], z_peer)
        sz.start()
        sz.wait()
        sx = rdma(st_all.at[pl.ds(xrow0, 2 * RB)],
                  st_all.at[pl.ds(xrow0, 2 * RB)],
                  ssend.at[2], srecv.at[2], x_peer)
        sx.start()
        sx.wait()

        def norm_region(rs, chalf):
            for q in range(CB // NQ):
                cs = chalf + q * NQ
                ld = pltpu.make_async_copy(
                    out_ref.at[pl.ds(rs, RB), pl.ds(cs, NQ)], stage, stage_sem)
                ld.start()
                ld.wait()
                mm = st_all[pl.ds(rs, RB), 0:1]
                iv = st_all[pl.ds(rs, RB), 128:129]
                stage[:, :] = jnp.exp(stage[:, :] - mm) * iv
                st = pltpu.make_async_copy(
                    stage, out_ref.at[pl.ds(rs, RB), pl.ds(cs, NQ)], stage_sem)
                st.start()
                st.wait()

        norm_region(row0, col0)

        for c in range(NC):
            z_h[c].wait_recv()
        norm_region(rowz, col0)
        y_b = rdma(out_ref.at[pl.ds(rowz, RB), pl.ds(col0, CB)],
                   out_ref.at[pl.ds(rowz, RB), pl.ds(col0, CB)],
                   fsend.at[0], frecv.at[0], y_peer)
        x_b = rdma(out_ref.at[pl.ds(rowz, RB), pl.ds(col0, CB)],
                   out_ref.at[pl.ds(rowz, RB), pl.ds(col0, CB)],
                   fsend.at[1], frecv.at[1], x_peer)
        y_b.start()
        x_b.start()

        for c in range(NC):
            y_h[c].wait_recv()
        norm_region(row0, colo)
        x_ca = rdma(out_ref.at[pl.ds(row0, RB), pl.ds(colo, CB)],
                    out_ref.at[pl.ds(row0, RB), pl.ds(colo, CB)],
                    fsend.at[2], frecv.at[2], x_peer)
        x_ca.start()

        for c in range(NC):
            x_h[c].wait_recv()
        norm_region(rowx0, col0)

        y_b.wait_recv()
        x_cb = rdma(out_ref.at[pl.ds(rowz, RB), pl.ds(colo, CB)],
                    out_ref.at[pl.ds(rowz, RB), pl.ds(colo, CB)],
                    fsend.at[3], frecv.at[3], x_peer)
        x_cb.start()

        x_b.wait_recv()
        x_ca.wait_recv()
        x_cb.wait_recv()
        y_b.wait_send()
        x_b.wait_send()
        x_ca.wait_send()
        x_cb.wait_send()

    return pl.pallas_call(
        body,
        out_shape=jax.ShapeDtypeStruct((T, V), jnp.float32),
        in_specs=[pl.BlockSpec(memory_space=pltpu.MemorySpace.VMEM),
                  pl.BlockSpec(memory_space=pl.ANY)],
        out_specs=pl.BlockSpec(memory_space=pltpu.MemorySpace.HBM),
        scratch_shapes=[
            pltpu.VMEM((2, D, WC), jnp.float32),
            pltpu.VMEM((PB, RB, WC), jnp.float32),
            pltpu.VMEM((RB, NQ), jnp.float32),
            pltpu.VMEM((RB, 256), jnp.float32),
            pltpu.VMEM((RB, 256), jnp.float32),
            pltpu.VMEM((T, 256), jnp.float32),
            pltpu.SemaphoreType.DMA((2,)),
            pltpu.SemaphoreType.DMA((NC,)),
            pltpu.SemaphoreType.DMA((NC,)),
            pltpu.SemaphoreType.DMA((NC,)),
            pltpu.SemaphoreType.DMA((NC,)),
            pltpu.SemaphoreType.DMA((NC,)),
            pltpu.SemaphoreType.DMA((NC,)),
            pltpu.SemaphoreType.DMA((NC,)),
            pltpu.SemaphoreType.DMA((4,)),
            pltpu.SemaphoreType.DMA((4,)),
            pltpu.SemaphoreType.DMA((3,)),
            pltpu.SemaphoreType.DMA((3,)),
            pltpu.SemaphoreType.DMA,
        ],
        compiler_params=pltpu.CompilerParams(
            collective_id=0, vmem_limit_bytes=60 * 1024 * 1024),
    )(x_rows, W)


# device time: 889754 ns/iter; 1.0101x vs baseline; 1.0004x over previous
import jax
import jax.numpy as jnp
from jax import lax
from jax.experimental import pallas as pl
from jax.experimental.pallas import tpu as pltpu

jax.config.update("jax_compilation_cache_dir", "/tmp/jax_comp_cache")
jax.config.update("jax_persistent_cache_min_compile_time_secs", 1.0)

T, D, V = 2048, 4096, 16384
RB = T // 4
CB = V // 2
NC = 16
WC = CB // NC
PB = 8
NQ = 4096


def kernel(x, W):
    xi = lax.axis_index("x")
    zi = lax.axis_index("z")
    x_rows = lax.dynamic_slice_in_dim(x, (2 * xi + zi) * RB, RB, axis=0)

    def body(x_ref, w_ref, out_ref, wbuf, piece, stage, st_mine, st_peer,
             st_all, wsems, lsems, zsend, zrecv, ysend, yrecv, xsend, xrecv,
             fsend, frecv, ssend, srecv, stage_sem):
        xi = lax.axis_index("x")
        yi = lax.axis_index("y")
        zi = lax.axis_index("z")
        row0 = (2 * xi + zi) * RB
        rowz = (2 * xi + (1 - zi)) * RB
        rowx0 = (2 * (1 - xi) + zi) * RB
        xrow0 = xi * (2 * RB)
        col0 = yi * CB
        colo = (1 - yi) * CB
        z_peer = (xi, yi, 1 - zi)
        y_peer = (xi, 1 - yi, zi)
        x_peer = (1 - xi, yi, zi)

        def rdma(src, dst, ssem, rsem, peer):
            return pltpu.make_async_remote_copy(
                src_ref=src, dst_ref=dst, send_sem=ssem, recv_sem=rsem,
                device_id=peer, device_id_type=pl.DeviceIdType.MESH)

        def wload(c):
            return pltpu.make_async_copy(
                w_ref.at[:, pl.ds(c * WC, WC)], wbuf.at[c % 2],
                wsems.at[c % 2])

        wload(0).start()

        barrier = pltpu.get_barrier_semaphore()
        for nbr in [x_peer, y_peer, z_peer]:
            pl.semaphore_signal(barrier, inc=1, device_id=nbr,
                                device_id_type=pl.DeviceIdType.MESH)
        pl.semaphore_wait(barrier, 3)

        x_val = x_ref[:, :].astype(jnp.bfloat16)
        z_h = [None] * NC
        y_h = [None] * NC
        x_h = [None] * NC
        l_h = [None] * NC
        m_run = None
        s_run = None

        for c in range(NC):
            pltpu.make_async_copy(
                w_ref.at[:, pl.ds(c * WC, WC)], wbuf.at[c % 2],
                wsems.at[c % 2]).wait()
            if c + 1 < NC:
                wload(c + 1).start()
            if c >= PB:
                z_h[c - PB].wait_send()
                y_h[c - PB].wait_send()
                x_h[c - PB].wait_send()
                l_h[c - PB].wait()
            l_c = jnp.dot(x_val, wbuf[c % 2, :, :].astype(jnp.bfloat16),
                          preferred_element_type=jnp.float32)
            pb = c % PB
            piece[pb, :, :] = l_c
            mc = jnp.max(l_c, axis=1, keepdims=True)
            if c == 0:
                m_run = mc
                s_run = jnp.sum(jnp.exp(l_c - mc), axis=1, keepdims=True)
            else:
                m_new = jnp.maximum(m_run, mc)
                s_run = (s_run * jnp.exp(m_run - m_new)
                         + jnp.sum(jnp.exp(l_c - m_new), axis=1, keepdims=True))
                m_run = m_new
            dst = out_ref.at[pl.ds(row0, RB), pl.ds(col0 + c * WC, WC)]
            l_h[c] = pltpu.make_async_copy(piece.at[pb], dst, lsems.at[c])
            l_h[c].start()
            z_h[c] = rdma(piece.at[pb], dst, zsend.at[c], zrecv.at[c], z_peer)
            y_h[c] = rdma(piece.at[pb], dst, ysend.at[c], yrecv.at[c], y_peer)
            x_h[c] = rdma(piece.at[pb], dst, xsend.at[c], xrecv.at[c], x_peer)
            z_h[c].start()
            y_h[c].start()
            x_h[c].start()

        for c in range(NC - PB, NC):
            z_h[c].wait_send()
            y_h[c].wait_send()
            x_h[c].wait_send()
            l_h[c].wait()

        st_mine[:, 0:128] = jnp.broadcast_to(m_run, (RB, 128))
        st_mine[:, 128:256] = jnp.broadcast_to(s_run, (RB, 128))
        sy = rdma(st_mine, st_peer, ssend.at[0], srecv.at[0], y_peer)
        sy.start()
        sy.wait()
        m_o = st_peer[:, 0:1]
        s_o = st_peer[:, 128:129]
        m_g = jnp.maximum(m_run, m_o)
        s_g = s_run * jnp.exp(m_run - m_g) + s_o * jnp.exp(m_o - m_g)
        st_all[pl.ds(row0, RB), 0:128] = jnp.broadcast_to(m_g, (RB, 128))
        st_all[pl.ds(row0, RB), 128:256] = jnp.broadcast_to(1.0 / s_g, (RB, 128))
        sz = rdma(st_all.at[pl.ds(row0, RB)], st_all.at[pl.ds(row0, RB)],
                  ssend.at[1], srecv.at[1], z_peer)
        sz.start()
        sz.wait()
        sx = rdma(st_all.at[pl.ds(xrow0, 2 * RB)],
                  st_all.at[pl.ds(xrow0, 2 * RB)],
                  ssend.at[2], srecv.at[2], x_peer)
        sx.start()
        sx.wait()

        def norm_region(rs, chalf):
            for q in range(CB // NQ):
                cs = chalf + q * NQ
                ld = pltpu.make_async_copy(
                    out_ref.at[pl.ds(rs, RB), pl.ds(cs, NQ)], stage, stage_sem)
                ld.start()
                ld.wait()
                mm = st_all[pl.ds(rs, RB), 0:1]
                iv = st_all[pl.ds(rs, RB), 128:129]
                stage[:, :] = jnp.exp(stage[:, :] - mm) * iv
                st = pltpu.make_async_copy(
                    stage, out_ref.at[pl.ds(rs, RB), pl.ds(cs, NQ)], stage_sem)
                st.start()
                st.wait()

        norm_region(row0, col0)

        for c in range(NC):
            z_h[c].wait_recv()
        norm_region(rowz, col0)
        y_b = rdma(out_ref.at[pl.ds(rowz, RB), pl.ds(col0, CB)],
                   out_ref.at[pl.ds(rowz, RB), pl.ds(col0, CB)],
                   fsend.at[0], frecv.at[0], y_peer)
        x_b = rdma(out_ref.at[pl.ds(rowz, RB), pl.ds(col0, CB)],
                   out_ref.at[pl.ds(rowz, RB), pl.ds(col0, CB)],
                   fsend.at[1], frecv.at[1], x_peer)
        y_b.start()
        x_b.start()

        for c in range(NC):
            y_h[c].wait_recv()
        norm_region(row0, colo)
        x_ca = rdma(out_ref.at[pl.ds(row0, RB), pl.ds(colo, CB)],
                    out_ref.at[pl.ds(row0, RB), pl.ds(colo, CB)],
                    fsend.at[2], frecv.at[2], x_peer)
        x_ca.start()

        for c in range(NC):
            x_h[c].wait_recv()
        norm_region(rowx0, col0)

        y_b.wait_recv()
        x_cb = rdma(out_ref.at[pl.ds(rowz, RB), pl.ds(colo, CB)],
                    out_ref.at[pl.ds(rowz, RB), pl.ds(colo, CB)],
                    fsend.at[3], frecv.at[3], x_peer)
        x_cb.start()

        x_b.wait_recv()
        x_ca.wait_recv()
        x_cb.wait_recv()
        y_b.wait_send()
        x_b.wait_send()
        x_ca.wait_send()
        x_cb.wait_send()

    return pl.pallas_call(
        body,
        out_shape=jax.ShapeDtypeStruct((T, V), jnp.float32),
        in_specs=[pl.BlockSpec(memory_space=pltpu.MemorySpace.VMEM),
                  pl.BlockSpec(memory_space=pl.ANY)],
        out_specs=pl.BlockSpec(memory_space=pltpu.MemorySpace.HBM),
        scratch_shapes=[
            pltpu.VMEM((2, D, WC), jnp.float32),
            pltpu.VMEM((PB, RB, WC), jnp.float32),
            pltpu.VMEM((RB, NQ), jnp.float32),
            pltpu.VMEM((RB, 256), jnp.float32),
            pltpu.VMEM((RB, 256), jnp.float32),
            pltpu.VMEM((T, 256), jnp.float32),
            pltpu.SemaphoreType.DMA((2,)),
            pltpu.SemaphoreType.DMA((NC,)),
            pltpu.SemaphoreType.DMA((NC,)),
            pltpu.SemaphoreType.DMA((NC,)),
            pltpu.SemaphoreType.DMA((NC,)),
            pltpu.SemaphoreType.DMA((NC,)),
            pltpu.SemaphoreType.DMA((NC,)),
            pltpu.SemaphoreType.DMA((NC,)),
            pltpu.SemaphoreType.DMA((4,)),
            pltpu.SemaphoreType.DMA((4,)),
            pltpu.SemaphoreType.DMA((3,)),
            pltpu.SemaphoreType.DMA((3,)),
            pltpu.SemaphoreType.DMA,
        ],
        compiler_params=pltpu.CompilerParams(
            collective_id=0, vmem_limit_bytes=60 * 1024 * 1024),
    )(x_rows, W)
